# baseline (speedup 1.0000x reference)
import functools

import jax
import jax.numpy as jnp
from jax import lax
from jax.experimental import pallas as pl
from jax.experimental.pallas import tpu as pltpu
from jax.experimental.pallas import tpu_sc as plsc

NC = 2
NS = 16
LANES = 16
CP = 48
EB = 128
_SCATTER_ON = True
_COMPUTE_ON = True


def _tc_a_body(c, x_ref, wl_ref, bl_ref, wr_ref, br_ref, att_ref,
               wlin_ref, blin_ref, xlp_ref, xrp_ref, lm_ref, self_ref):
    bn = x_ref.shape[0]
    xb = x_ref[...]
    xl = jnp.dot(xb, wl_ref[...], preferred_element_type=jnp.float32) + bl_ref[...]
    xr = jnp.dot(xb, wr_ref[...], preferred_element_type=jnp.float32) + br_ref[...]
    lm_ref[...] = (jnp.dot(xb, wlin_ref[...], preferred_element_type=jnp.float32)
                   + blin_ref[...])
    one = jnp.ones((bn, 1), jnp.float32)
    zpad = jnp.zeros((bn, CP - c - 1), jnp.float32)
    xlp = jnp.concatenate([xl, one, zpad], axis=1)
    xlp_ref[...] = xlp
    xrp_ref[...] = jnp.concatenate(
        [xr, jnp.zeros((bn, CP - c), jnp.float32)], axis=1)
    u = xl + xr
    s = jnp.maximum(u, 0.2 * u)
    logit = jnp.sum(s * att_ref[...], axis=1, keepdims=True)
    self_ref[...] = xlp * jnp.exp(logit)


def _tc_a(x, w_l, b_l, w_r, b_r, att, w_lin, b_lin):
    n, f = x.shape
    c = w_l.shape[1]
    bn = 1000
    grid = (n // bn,)
    full = lambda shape: pl.BlockSpec(shape, lambda i: (0,) * len(shape))
    row = lambda shape: pl.BlockSpec(shape, lambda i: (i,) + (0,) * (len(shape) - 1))
    return pl.pallas_call(
        functools.partial(_tc_a_body, c),
        grid=grid,
        in_specs=[
            row((bn, f)), full((f, c)), full((1, c)), full((f, c)),
            full((1, c)), full((1, c)), full((f, c)), full((1, c)),
        ],
        out_specs=[row((bn, CP)), row((bn, CP)), row((bn, c)), row((bn, CP))],
        out_shape=[
            jax.ShapeDtypeStruct((n, CP), jnp.float32),
            jax.ShapeDtypeStruct((n, CP), jnp.float32),
            jax.ShapeDtypeStruct((n, c), jnp.float32),
            jax.ShapeDtypeStruct((n, CP), jnp.float32),
        ],
    )(x, w_l, b_l.reshape(1, c), w_r, b_r.reshape(1, c),
      att.reshape(1, c), w_lin, b_lin.reshape(1, c))


def _sc_edge_body(c, n, nblocks, xl_hbm, xr_hbm, src_hbm, dst_hbm, att_hbm,
                  out_hbm, src_v, dst_v, rows_l, rows_r, out_rows,
                  att_v, zbuf, acc,
                  sem_l0, sem_l1, sem_r0, sem_r1, sem_s0, sem_s1):
    cid = lax.axis_index("c")
    sid = lax.axis_index("s")
    wid = sid * NC + cid
    nw = NC * NS
    rb = -(-nblocks // nw)
    rb_last = nblocks - rb * (nw - 1)
    rows_per_sub = (n // 8 // NS) * 8
    tail = n - rows_per_sub * NS

    zvec = jnp.zeros((LANES,), jnp.float32)

    def zero_body(i, _):
        r = i // (CP // LANES)
        col = (i % (CP // LANES)) * LANES
        zbuf[r, pl.ds(col, LANES)] = zvec
        return 0

    lax.fori_loop(0, rows_per_sub * (CP // LANES), zero_body, 0)
    pltpu.sync_copy(zbuf, acc.at[pl.ds(sid * rows_per_sub, rows_per_sub)])
    if tail:
        @pl.when(sid == NS - 1)
        def _():
            pltpu.sync_copy(zbuf.at[pl.ds(0, tail)],
                            acc.at[pl.ds(rows_per_sub * NS, tail)])

    row0 = wid * rb

    @pl.when(wid < nw - 1)
    def _():
        pltpu.sync_copy(src_hbm.at[pl.ds(row0, rb)], src_v)
        pltpu.sync_copy(dst_hbm.at[pl.ds(row0, rb)], dst_v)

    @pl.when(wid == nw - 1)
    def _():
        pltpu.sync_copy(src_hbm.at[pl.ds(row0, rb_last)],
                        src_v.at[pl.ds(0, rb_last)])
        pltpu.sync_copy(dst_hbm.at[pl.ds(row0, rb_last)],
                        dst_v.at[pl.ds(0, rb_last)])

    nb_w = jnp.where(wid == nw - 1, rb_last, rb)

    pltpu.sync_copy(att_hbm, att_v)

    plsc.subcore_barrier()

    iota = lax.iota(jnp.int32, LANES)
    slots = ((rows_l.at[0], rows_r.at[0], out_rows.at[0], sem_l0, sem_r0,
              sem_s0),
             (rows_l.at[1], rows_r.at[1], out_rows.at[1], sem_l1, sem_r1,
              sem_s1))

    def gathers(b, s):
        rl, rr = slots[s][0], slots[s][1]
        cl = pltpu.make_async_copy(xl_hbm.at[src_v.at[b]], rl, slots[s][3])
        cr = pltpu.make_async_copy(xr_hbm.at[dst_v.at[b]], rr, slots[s][4])
        return cl, cr

    def scatter(b, s):
        return pltpu.make_async_copy(slots[s][2], acc.at[dst_v.at[b]],
                                     slots[s][5])

    def compute(s):
        rl, rr, orw = slots[s][0], slots[s][1], slots[s][2]
        av = [att_v[pl.ds(k * LANES, LANES)] for k in range(CP // LANES)]

        def group_body(g, _):
            for j in range(LANES):
                e = g * LANES + j
                lv = [rl[e, pl.ds(k * LANES, LANES)]
                      for k in range(CP // LANES)]
                w = jnp.zeros((LANES,), jnp.float32)
                for k in range(CP // LANES):
                    u = lv[k] + rr[e, pl.ds(k * LANES, LANES)]
                    su = jnp.maximum(u, 0.2 * u)
                    w = w + av[k] * su
                logit = jnp.sum(w)
                tb = jnp.exp(jnp.broadcast_to(logit, (LANES,)))
                for k in range(CP // LANES):
                    orw[e, pl.ds(k * LANES, LANES)] = lv[k] * tb
            return 0

        lax.fori_loop(0, EB // LANES, group_body, 0)

    cl, cr = gathers(0, 0)
    cl.start()
    cr.start()

    def slot_step(b, s, p):
        @pl.when(b < nb_w)
        def _():
            cl, cr = gathers(b, s)
            cl.wait()
            cr.wait()

            @pl.when(b + 1 < nb_w)
            def _():
                nl, nr = gathers(b + 1, 1 - s)
                nl.start()
                nr.start()

            if _SCATTER_ON:
                @pl.when(p >= 1)
                def _():
                    scatter(b, s).wait()

            if _COMPUTE_ON:
                compute(s)
            if _SCATTER_ON:
                sc = scatter(b, s)
                sc.start(add=True)

    def pair_body(p, _):
        slot_step(2 * p, 0, p)
        slot_step(2 * p + 1, 1, p)
        return 0

    lax.fori_loop(0, (rb + 1) // 2, pair_body, 0)

    if _SCATTER_ON:
        scatter(0, 0).wait()
        scatter(0, 1).wait()

    plsc.subcore_barrier()

    r0 = sid * rows_per_sub
    pltpu.sync_copy(acc.at[pl.ds(r0, rows_per_sub)],
                    out_hbm.at[cid, pl.ds(r0, rows_per_sub)])
    if tail:
        @pl.when(sid == NS - 1)
        def _():
            pltpu.sync_copy(acc.at[pl.ds(rows_per_sub * NS, tail)],
                            out_hbm.at[cid, pl.ds(rows_per_sub * NS, tail)])


def _sc_edge(xl_pad, xr_pad, src2d, dst2d, att_pad):
    n = xl_pad.shape[0]
    c = 40
    nblocks = src2d.shape[0]
    rb = -(-nblocks // (NC * NS))
    rows_per_sub = (n // 8 // NS) * 8
    mesh = plsc.VectorSubcoreMesh(core_axis_name="c", subcore_axis_name="s",
                                  num_cores=NC, num_subcores=NS)
    kern = functools.partial(
        pl.kernel,
        out_type=jax.ShapeDtypeStruct((NC, n, CP), jnp.float32),
        mesh=mesh,
        compiler_params=pltpu.CompilerParams(needs_layout_passes=False,
                                             use_tc_tiling_on_sc=False),
        scratch_types=[
            pltpu.VMEM((rb, EB), jnp.int32),
            pltpu.VMEM((rb, EB), jnp.int32),
            pltpu.VMEM((2, EB, CP), jnp.float32),
            pltpu.VMEM((2, EB, CP), jnp.float32),
            pltpu.VMEM((2, EB, CP), jnp.float32),
            pltpu.VMEM((CP,), jnp.float32),
            pltpu.VMEM((rows_per_sub, CP), jnp.float32),
            pltpu.VMEM_SHARED((n, CP), jnp.float32),
            pltpu.SemaphoreType.DMA,
            pltpu.SemaphoreType.DMA,
            pltpu.SemaphoreType.DMA,
            pltpu.SemaphoreType.DMA,
            pltpu.SemaphoreType.DMA,
            pltpu.SemaphoreType.DMA,
        ],
    )(functools.partial(_sc_edge_body, c, n, nblocks))
    return kern(xl_pad, xr_pad, src2d, dst2d, att_pad)


def _tc_b_body(c, p_ref, self_ref, bias_ref, out_ref):
    tot = p_ref[0] + p_ref[1] + self_ref[...]
    den = tot[:, c:c + 1] + 1e-16
    out_ref[...] = tot[:, :c] / den + bias_ref[...]


def _tc_b(partials, self_rows, bias_gat):
    _, n, _ = partials.shape
    c = bias_gat.shape[0]
    bn = 1000
    return pl.pallas_call(
        functools.partial(_tc_b_body, c),
        grid=(n // bn,),
        in_specs=[
            pl.BlockSpec((NC, bn, CP), lambda i: (0, i, 0)),
            pl.BlockSpec((bn, CP), lambda i: (i, 0)),
            pl.BlockSpec((1, c), lambda i: (0, 0)),
        ],
        out_specs=pl.BlockSpec((bn, c), lambda i: (i, 0)),
        out_shape=jax.ShapeDtypeStruct((n, c), jnp.float32),
    )(partials, self_rows, bias_gat.reshape(1, c))


def kernel(x, edge_index, W_l, b_l, W_r, b_r, att, bias_gat, W_lin, b_lin):
    c = W_l.shape[1]
    xl_pad, xr_pad, out_lm, self_rows = _tc_a(
        x, W_l, b_l, W_r, b_r, att, W_lin, b_lin)
    att_pad = jnp.concatenate([att, jnp.zeros((CP - c,), jnp.float32)])
    e = edge_index.shape[1]
    assert e % EB == 0
    src2d = edge_index[0].reshape(e // EB, EB)
    dst2d = edge_index[1].reshape(e // EB, EB)
    partials = _sc_edge(xl_pad, xr_pad, src2d, dst2d, att_pad)
    out_gnn = _tc_b(partials, self_rows, bias_gat)
    return (out_gnn, out_lm, x)


# speedup vs baseline: 1.0321x; 1.0321x over previous
import functools

import jax
import jax.numpy as jnp
from jax import lax
from jax.experimental import pallas as pl
from jax.experimental.pallas import tpu as pltpu
from jax.experimental.pallas import tpu_sc as plsc

NC = 2
NS = 16
LANES = 16
CP = 48
EB = 128
NBUF = 3
ZR = 104


def _tc_a_body(c, x_ref, wl_ref, bl_ref, wr_ref, br_ref, att_ref,
               wlin_ref, blin_ref, xlp_ref, xrp_ref, lm_ref, self_ref):
    bn = x_ref.shape[0]
    xb = x_ref[...]
    xl = jnp.dot(xb, wl_ref[...], preferred_element_type=jnp.float32) + bl_ref[...]
    xr = jnp.dot(xb, wr_ref[...], preferred_element_type=jnp.float32) + br_ref[...]
    lm_ref[...] = (jnp.dot(xb, wlin_ref[...], preferred_element_type=jnp.float32)
                   + blin_ref[...])
    one = jnp.ones((bn, 1), jnp.float32)
    zpad = jnp.zeros((bn, CP - c - 1), jnp.float32)
    xlp = jnp.concatenate([xl, one, zpad], axis=1)
    xlp_ref[...] = xlp
    xrp_ref[...] = jnp.concatenate(
        [xr, jnp.zeros((bn, CP - c), jnp.float32)], axis=1)
    u = xl + xr
    s = jnp.maximum(u, 0.2 * u)
    logit = jnp.sum(s * att_ref[...], axis=1, keepdims=True)
    self_ref[...] = xlp * jnp.exp(logit)


def _tc_a(x, w_l, b_l, w_r, b_r, att, w_lin, b_lin):
    n, f = x.shape
    c = w_l.shape[1]
    bn = 1000
    grid = (n // bn,)
    full = lambda shape: pl.BlockSpec(shape, lambda i: (0,) * len(shape))
    row = lambda shape: pl.BlockSpec(shape, lambda i: (i,) + (0,) * (len(shape) - 1))
    return pl.pallas_call(
        functools.partial(_tc_a_body, c),
        grid=grid,
        in_specs=[
            row((bn, f)), full((f, c)), full((1, c)), full((f, c)),
            full((1, c)), full((1, c)), full((f, c)), full((1, c)),
        ],
        out_specs=[row((bn, CP)), row((bn, CP)), row((bn, c)), row((bn, CP))],
        out_shape=[
            jax.ShapeDtypeStruct((n, CP), jnp.float32),
            jax.ShapeDtypeStruct((n, CP), jnp.float32),
            jax.ShapeDtypeStruct((n, c), jnp.float32),
            jax.ShapeDtypeStruct((n, CP), jnp.float32),
        ],
    )(x, w_l, b_l.reshape(1, c), w_r, b_r.reshape(1, c),
      att.reshape(1, c), w_lin, b_lin.reshape(1, c))


def _sc_edge_body(c, n, nblocks, xl_hbm, xr_hbm, src_hbm, dst_hbm, att_hbm,
                  out_hbm, src_v, dst_v, rows_l, rows_r, out_rows,
                  att_v, zbuf, acc, *all_sems):
    cid = lax.axis_index("c")
    sid = lax.axis_index("s")
    wid = sid * NC + cid
    nw = NC * NS
    rb = -(-nblocks // nw)
    rb_last = nblocks - rb * (nw - 1)
    rows_per_sub = (n // 8 // NS) * 8
    tail = n - rows_per_sub * NS

    zvec = jnp.zeros((LANES,), jnp.float32)

    def zero_body(i, _):
        r = i // (CP // LANES)
        col = (i % (CP // LANES)) * LANES
        zbuf[r, pl.ds(col, LANES)] = zvec
        return 0

    lax.fori_loop(0, ZR * (CP // LANES), zero_body, 0)
    nz = rows_per_sub // ZR

    def zcopy_body(i, _):
        pltpu.sync_copy(
            zbuf, acc.at[pl.ds(sid * rows_per_sub + i * ZR, ZR)])
        return 0

    lax.fori_loop(0, nz, zcopy_body, 0)
    if rows_per_sub % ZR:
        pltpu.sync_copy(
            zbuf.at[pl.ds(0, rows_per_sub % ZR)],
            acc.at[pl.ds(sid * rows_per_sub + nz * ZR, rows_per_sub % ZR)])
    if tail:
        @pl.when(sid == NS - 1)
        def _():
            pltpu.sync_copy(zbuf.at[pl.ds(0, tail)],
                            acc.at[pl.ds(rows_per_sub * NS, tail)])

    row0 = wid * rb

    @pl.when(wid < nw - 1)
    def _():
        pltpu.sync_copy(src_hbm.at[pl.ds(row0, rb)], src_v)
        pltpu.sync_copy(dst_hbm.at[pl.ds(row0, rb)], dst_v)

    @pl.when(wid == nw - 1)
    def _():
        pltpu.sync_copy(src_hbm.at[pl.ds(row0, rb_last)],
                        src_v.at[pl.ds(0, rb_last)])
        pltpu.sync_copy(dst_hbm.at[pl.ds(row0, rb_last)],
                        dst_v.at[pl.ds(0, rb_last)])

    nb_w = jnp.where(wid == nw - 1, rb_last, rb)

    pltpu.sync_copy(att_hbm, att_v)

    plsc.subcore_barrier()

    iota = lax.iota(jnp.int32, LANES)
    slots = tuple(
        (rows_l.at[i], rows_r.at[i], out_rows.at[i])
        + tuple(all_sems[3 * i:3 * i + 3])
        for i in range(NBUF))

    def gathers(b, s):
        rl, rr = slots[s][0], slots[s][1]
        cl = pltpu.make_async_copy(xl_hbm.at[src_v.at[b]], rl, slots[s][3])
        cr = pltpu.make_async_copy(xr_hbm.at[dst_v.at[b]], rr, slots[s][4])
        return cl, cr

    def scatter(b, s):
        return pltpu.make_async_copy(slots[s][2], acc.at[dst_v.at[b]],
                                     slots[s][5])

    def compute(s):
        rl, rr, orw = slots[s][0], slots[s][1], slots[s][2]
        av = [att_v[pl.ds(k * LANES, LANES)] for k in range(CP // LANES)]

        def group_body(g, _):
            for j in range(LANES):
                e = g * LANES + j
                lv = [rl[e, pl.ds(k * LANES, LANES)]
                      for k in range(CP // LANES)]
                w = jnp.zeros((LANES,), jnp.float32)
                for k in range(CP // LANES):
                    u = lv[k] + rr[e, pl.ds(k * LANES, LANES)]
                    su = jnp.maximum(u, 0.2 * u)
                    w = w + av[k] * su
                logit = jnp.sum(w)
                tb = jnp.exp(jnp.broadcast_to(logit, (LANES,)))
                for k in range(CP // LANES):
                    orw[e, pl.ds(k * LANES, LANES)] = lv[k] * tb
            return 0

        lax.fori_loop(0, EB // LANES, group_body, 0)

    for s0 in range(NBUF - 1):
        @pl.when(s0 < nb_w)
        def _(s0=s0):
            cl, cr = gathers(s0, s0)
            cl.start()
            cr.start()

    def slot_step(b, s, p):
        @pl.when(b < nb_w)
        def _():
            cl, cr = gathers(b, s)
            cl.wait()
            cr.wait()

            @pl.when(b + NBUF - 1 < nb_w)
            def _():
                nl, nr = gathers(b + NBUF - 1, (s + NBUF - 1) % NBUF)
                nl.start()
                nr.start()

            @pl.when(p >= 1)
            def _():
                scatter(b, s).wait()

            compute(s)
            sc = scatter(b, s)
            sc.start(add=True)

    def ring_body(p, _):
        for s in range(NBUF):
            slot_step(NBUF * p + s, s, p)
        return 0

    lax.fori_loop(0, -(-rb // NBUF), ring_body, 0)

    for s in range(NBUF):
        scatter(0, s).wait()

    plsc.subcore_barrier()

    r0 = sid * rows_per_sub
    pltpu.sync_copy(acc.at[pl.ds(r0, rows_per_sub)],
                    out_hbm.at[cid, pl.ds(r0, rows_per_sub)])
    if tail:
        @pl.when(sid == NS - 1)
        def _():
            pltpu.sync_copy(acc.at[pl.ds(rows_per_sub * NS, tail)],
                            out_hbm.at[cid, pl.ds(rows_per_sub * NS, tail)])


def _sc_edge(xl_pad, xr_pad, src2d, dst2d, att_pad):
    n = xl_pad.shape[0]
    c = 40
    nblocks = src2d.shape[0]
    rb = -(-nblocks // (NC * NS))
    rows_per_sub = (n // 8 // NS) * 8
    mesh = plsc.VectorSubcoreMesh(core_axis_name="c", subcore_axis_name="s",
                                  num_cores=NC, num_subcores=NS)
    kern = functools.partial(
        pl.kernel,
        out_type=jax.ShapeDtypeStruct((NC, n, CP), jnp.float32),
        mesh=mesh,
        compiler_params=pltpu.CompilerParams(needs_layout_passes=False,
                                             use_tc_tiling_on_sc=False),
        scratch_types=[
            pltpu.VMEM((rb, EB), jnp.int32),
            pltpu.VMEM((rb, EB), jnp.int32),
            pltpu.VMEM((NBUF, EB, CP), jnp.float32),
            pltpu.VMEM((NBUF, EB, CP), jnp.float32),
            pltpu.VMEM((NBUF, EB, CP), jnp.float32),
            pltpu.VMEM((CP,), jnp.float32),
            pltpu.VMEM((ZR, CP), jnp.float32),
            pltpu.VMEM_SHARED((n, CP), jnp.float32),
        ] + [pltpu.SemaphoreType.DMA] * (3 * NBUF),
    )(functools.partial(_sc_edge_body, c, n, nblocks))
    return kern(xl_pad, xr_pad, src2d, dst2d, att_pad)


def _tc_b_body(c, p_ref, self_ref, bias_ref, out_ref):
    tot = p_ref[0] + p_ref[1] + self_ref[...]
    den = tot[:, c:c + 1] + 1e-16
    out_ref[...] = tot[:, :c] / den + bias_ref[...]


def _tc_b(partials, self_rows, bias_gat):
    _, n, _ = partials.shape
    c = bias_gat.shape[0]
    bn = 1000
    return pl.pallas_call(
        functools.partial(_tc_b_body, c),
        grid=(n // bn,),
        in_specs=[
            pl.BlockSpec((NC, bn, CP), lambda i: (0, i, 0)),
            pl.BlockSpec((bn, CP), lambda i: (i, 0)),
            pl.BlockSpec((1, c), lambda i: (0, 0)),
        ],
        out_specs=pl.BlockSpec((bn, c), lambda i: (i, 0)),
        out_shape=jax.ShapeDtypeStruct((n, c), jnp.float32),
    )(partials, self_rows, bias_gat.reshape(1, c))


def kernel(x, edge_index, W_l, b_l, W_r, b_r, att, bias_gat, W_lin, b_lin):
    c = W_l.shape[1]
    xl_pad, xr_pad, out_lm, self_rows = _tc_a(
        x, W_l, b_l, W_r, b_r, att, W_lin, b_lin)
    att_pad = jnp.concatenate([att, jnp.zeros((CP - c,), jnp.float32)])
    e = edge_index.shape[1]
    assert e % EB == 0
    src2d = edge_index[0].reshape(e // EB, EB)
    dst2d = edge_index[1].reshape(e // EB, EB)
    partials = _sc_edge(xl_pad, xr_pad, src2d, dst2d, att_pad)
    out_gnn = _tc_b(partials, self_rows, bias_gat)
    return (out_gnn, out_lm, x)


# speedup vs baseline: 1.0816x; 1.0480x over previous
import functools

import jax
import jax.numpy as jnp
from jax import lax
from jax.experimental import pallas as pl
from jax.experimental.pallas import tpu as pltpu
from jax.experimental.pallas import tpu_sc as plsc

NC = 2
NS = 16
LANES = 16
CP = 48
EB = 128
NBUF = 3
ZR = 104


def _tc_a1_body(c, x_ref, wl_ref, bl_ref, wr_ref, br_ref, att_ref,
                xlp_ref, xrp_ref, attp_ref):
    bn = x_ref.shape[0]
    xb = x_ref[...]
    xl = jnp.dot(xb, wl_ref[...], preferred_element_type=jnp.float32) + bl_ref[...]
    xr = jnp.dot(xb, wr_ref[...], preferred_element_type=jnp.float32) + br_ref[...]
    one = jnp.ones((bn, 1), jnp.float32)
    zpad = jnp.zeros((bn, CP - c - 1), jnp.float32)
    xlp_ref[...] = jnp.concatenate([xl, one, zpad], axis=1)
    xrp_ref[...] = jnp.concatenate(
        [xr, jnp.zeros((bn, CP - c), jnp.float32)], axis=1)
    attp_ref[...] = jnp.concatenate(
        [att_ref[...], jnp.zeros((1, CP - c), jnp.float32)], axis=1)


def _tc_a1(x, w_l, b_l, w_r, b_r, att):
    n, f = x.shape
    c = w_l.shape[1]
    bn = 1000
    full = lambda shape: pl.BlockSpec(shape, lambda i: (0,) * len(shape))
    row = lambda shape: pl.BlockSpec(shape, lambda i: (i,) + (0,) * (len(shape) - 1))
    return pl.pallas_call(
        functools.partial(_tc_a1_body, c),
        grid=(n // bn,),
        in_specs=[
            row((bn, f)), full((f, c)), full((1, c)), full((f, c)),
            full((1, c)), full((1, c)),
        ],
        out_specs=[row((bn, CP)), row((bn, CP)), full((1, CP))],
        out_shape=[
            jax.ShapeDtypeStruct((n, CP), jnp.float32),
            jax.ShapeDtypeStruct((n, CP), jnp.float32),
            jax.ShapeDtypeStruct((1, CP), jnp.float32),
        ],
    )(x, w_l, b_l.reshape(1, c), w_r, b_r.reshape(1, c), att.reshape(1, c))


def _tc_a2_body(c, x_ref, xlp_ref, xrp_ref, att_ref, wlin_ref, blin_ref,
                lm_ref, self_ref):
    xb = x_ref[...]
    lm_ref[...] = (jnp.dot(xb, wlin_ref[...], preferred_element_type=jnp.float32)
                   + blin_ref[...])
    xlp = xlp_ref[...]
    xl = xlp[:, :c]
    xr = xrp_ref[:, :c]
    u = xl + xr
    s = jnp.maximum(u, 0.2 * u)
    logit = jnp.sum(s * att_ref[...], axis=1, keepdims=True)
    self_ref[...] = xlp * jnp.exp(logit)


def _tc_a2(x, xl_pad, xr_pad, att, w_lin, b_lin):
    n, f = x.shape
    c = w_lin.shape[1]
    bn = 1000
    full = lambda shape: pl.BlockSpec(shape, lambda i: (0,) * len(shape))
    row = lambda shape: pl.BlockSpec(shape, lambda i: (i,) + (0,) * (len(shape) - 1))
    return pl.pallas_call(
        functools.partial(_tc_a2_body, c),
        grid=(n // bn,),
        in_specs=[
            row((bn, f)), row((bn, CP)), row((bn, CP)), full((1, c)),
            full((f, c)), full((1, c)),
        ],
        out_specs=[row((bn, c)), row((bn, CP))],
        out_shape=[
            jax.ShapeDtypeStruct((n, c), jnp.float32),
            jax.ShapeDtypeStruct((n, CP), jnp.float32),
        ],
    )(x, xl_pad, xr_pad, att.reshape(1, c), w_lin, b_lin.reshape(1, c))


def _sc_edge_body(c, n, nblocks, xl_hbm, xr_hbm, edg_hbm, att_hbm,
                  out_hbm, src_v, dst_v, rows_l, rows_r, out_rows,
                  att_v, zbuf, acc, *all_sems):
    cid = lax.axis_index("c")
    sid = lax.axis_index("s")
    wid = sid * NC + cid
    nw = NC * NS
    rb = -(-nblocks // nw)
    rb_last = nblocks - rb * (nw - 1)
    rows_per_sub = (n // 8 // NS) * 8
    tail = n - rows_per_sub * NS

    zvec = jnp.zeros((LANES,), jnp.float32)

    def zero_body(i, _):
        r = i // (CP // LANES)
        col = (i % (CP // LANES)) * LANES
        zbuf[r, pl.ds(col, LANES)] = zvec
        return 0

    lax.fori_loop(0, ZR * (CP // LANES), zero_body, 0)
    nz = rows_per_sub // ZR

    def zcopy_body(i, _):
        pltpu.sync_copy(
            zbuf, acc.at[pl.ds(sid * rows_per_sub + i * ZR, ZR)])
        return 0

    lax.fori_loop(0, nz, zcopy_body, 0)
    if rows_per_sub % ZR:
        pltpu.sync_copy(
            zbuf.at[pl.ds(0, rows_per_sub % ZR)],
            acc.at[pl.ds(sid * rows_per_sub + nz * ZR, rows_per_sub % ZR)])
    if tail:
        @pl.when(sid == NS - 1)
        def _():
            pltpu.sync_copy(zbuf.at[pl.ds(0, tail)],
                            acc.at[pl.ds(rows_per_sub * NS, tail)])

    row0 = wid * rb

    @pl.when(wid < nw - 1)
    def _():
        pltpu.sync_copy(edg_hbm.at[0, pl.ds(row0, rb)], src_v)
        pltpu.sync_copy(edg_hbm.at[1, pl.ds(row0, rb)], dst_v)

    @pl.when(wid == nw - 1)
    def _():
        pltpu.sync_copy(edg_hbm.at[0, pl.ds(row0, rb_last)],
                        src_v.at[pl.ds(0, rb_last)])
        pltpu.sync_copy(edg_hbm.at[1, pl.ds(row0, rb_last)],
                        dst_v.at[pl.ds(0, rb_last)])

    nb_w = jnp.where(wid == nw - 1, rb_last, rb)

    pltpu.sync_copy(att_hbm.at[0], att_v)

    plsc.subcore_barrier()

    iota = lax.iota(jnp.int32, LANES)
    slots = tuple(
        (rows_l.at[i], rows_r.at[i], out_rows.at[i])
        + tuple(all_sems[3 * i:3 * i + 3])
        for i in range(NBUF))

    def gathers(b, s):
        rl, rr = slots[s][0], slots[s][1]
        cl = pltpu.make_async_copy(xl_hbm.at[src_v.at[b]], rl, slots[s][3])
        cr = pltpu.make_async_copy(xr_hbm.at[dst_v.at[b]], rr, slots[s][4])
        return cl, cr

    def scatter(b, s):
        return pltpu.make_async_copy(slots[s][2], acc.at[dst_v.at[b]],
                                     slots[s][5])

    def compute(s):
        rl, rr, orw = slots[s][0], slots[s][1], slots[s][2]
        av = [att_v[pl.ds(k * LANES, LANES)] for k in range(CP // LANES)]

        def group_body(g, _):
            for j in range(LANES):
                e = g * LANES + j
                lv = [rl[e, pl.ds(k * LANES, LANES)]
                      for k in range(CP // LANES)]
                w = jnp.zeros((LANES,), jnp.float32)
                for k in range(CP // LANES):
                    u = lv[k] + rr[e, pl.ds(k * LANES, LANES)]
                    su = jnp.maximum(u, 0.2 * u)
                    w = w + av[k] * su
                logit = jnp.sum(w)
                tb = jnp.exp(jnp.broadcast_to(logit, (LANES,)))
                for k in range(CP // LANES):
                    orw[e, pl.ds(k * LANES, LANES)] = lv[k] * tb
            return 0

        lax.fori_loop(0, EB // LANES, group_body, 0)

    for s0 in range(NBUF - 1):
        @pl.when(s0 < nb_w)
        def _(s0=s0):
            cl, cr = gathers(s0, s0)
            cl.start()
            cr.start()

    def slot_step(b, s, p):
        @pl.when(b < nb_w)
        def _():
            cl, cr = gathers(b, s)
            cl.wait()
            cr.wait()

            @pl.when(b + NBUF - 1 < nb_w)
            def _():
                nl, nr = gathers(b + NBUF - 1, (s + NBUF - 1) % NBUF)
                nl.start()
                nr.start()

            @pl.when(p >= 1)
            def _():
                scatter(b, s).wait()

            compute(s)
            sc = scatter(b, s)
            sc.start(add=True)

    def ring_body(p, _):
        for s in range(NBUF):
            slot_step(NBUF * p + s, s, p)
        return 0

    lax.fori_loop(0, -(-rb // NBUF), ring_body, 0)

    for s in range(NBUF):
        scatter(0, s).wait()

    plsc.subcore_barrier()

    r0 = sid * rows_per_sub
    pltpu.sync_copy(acc.at[pl.ds(r0, rows_per_sub)],
                    out_hbm.at[cid, pl.ds(r0, rows_per_sub)])
    if tail:
        @pl.when(sid == NS - 1)
        def _():
            pltpu.sync_copy(acc.at[pl.ds(rows_per_sub * NS, tail)],
                            out_hbm.at[cid, pl.ds(rows_per_sub * NS, tail)])


def _sc_edge(xl_pad, xr_pad, edg3d, att_pad):
    n = xl_pad.shape[0]
    c = 40
    nblocks = edg3d.shape[1]
    rb = -(-nblocks // (NC * NS))
    rows_per_sub = (n // 8 // NS) * 8
    mesh = plsc.VectorSubcoreMesh(core_axis_name="c", subcore_axis_name="s",
                                  num_cores=NC, num_subcores=NS)
    kern = functools.partial(
        pl.kernel,
        out_type=jax.ShapeDtypeStruct((NC, n, CP), jnp.float32),
        mesh=mesh,
        compiler_params=pltpu.CompilerParams(needs_layout_passes=False,
                                             use_tc_tiling_on_sc=False),
        scratch_types=[
            pltpu.VMEM((rb, EB), jnp.int32),
            pltpu.VMEM((rb, EB), jnp.int32),
            pltpu.VMEM((NBUF, EB, CP), jnp.float32),
            pltpu.VMEM((NBUF, EB, CP), jnp.float32),
            pltpu.VMEM((NBUF, EB, CP), jnp.float32),
            pltpu.VMEM((CP,), jnp.float32),
            pltpu.VMEM((ZR, CP), jnp.float32),
            pltpu.VMEM_SHARED((n, CP), jnp.float32),
        ] + [pltpu.SemaphoreType.DMA] * (3 * NBUF),
    )(functools.partial(_sc_edge_body, c, n, nblocks))
    return kern(xl_pad, xr_pad, edg3d, att_pad)


def _tc_b_body(c, p_ref, self_ref, bias_ref, out_ref):
    tot = p_ref[0] + p_ref[1] + self_ref[...]
    den = tot[:, c:c + 1] + 1e-16
    out_ref[...] = tot[:, :c] / den + bias_ref[...]


def _tc_b(partials, self_rows, bias_gat):
    _, n, _ = partials.shape
    c = bias_gat.shape[0]
    bn = 1000
    return pl.pallas_call(
        functools.partial(_tc_b_body, c),
        grid=(n // bn,),
        in_specs=[
            pl.BlockSpec((NC, bn, CP), lambda i: (0, i, 0)),
            pl.BlockSpec((bn, CP), lambda i: (i, 0)),
            pl.BlockSpec((1, c), lambda i: (0, 0)),
        ],
        out_specs=pl.BlockSpec((bn, c), lambda i: (i, 0)),
        out_shape=jax.ShapeDtypeStruct((n, c), jnp.float32),
    )(partials, self_rows, bias_gat.reshape(1, c))


def kernel(x, edge_index, W_l, b_l, W_r, b_r, att, bias_gat, W_lin, b_lin):
    xl_pad, xr_pad, att_pad = _tc_a1(x, W_l, b_l, W_r, b_r, att)
    e = edge_index.shape[1]
    assert e % EB == 0
    edg3d = edge_index.reshape(2, e // EB, EB)
    partials = _sc_edge(xl_pad, xr_pad, edg3d, att_pad)
    out_lm, self_rows = _tc_a2(x, xl_pad, xr_pad, att, W_lin, b_lin)
    out_gnn = _tc_b(partials, self_rows, bias_gat)
    return (out_gnn, out_lm, x)
